# baseline (speedup 1.0000x reference)
import jax
import jax.numpy as jnp
from jax.experimental import pallas as pl
from jax.experimental.pallas import tpu as pltpu

_TILE = 2048
_ROLE_W = 0


def _layernorm(x, g, b, eps=1e-5):
    m = jnp.mean(x, axis=-1, keepdims=True)
    c = x - m
    v = jnp.mean(c * c, axis=-1, keepdims=True)
    return c * jax.lax.rsqrt(v + eps) * g + b


def _fused(rk_ref, aux_ref, res_ref, role_ref, Wk_ref, Wa_ref, Wr_ref,
           b0_ref, gin_ref, bin_ref, W1_ref, b1_ref, g1_ref, bb1_ref,
           W2_ref, b2_ref, g2_ref, bb2_ref, Whd_ref, bhd_ref, out_ref):
    f32 = jnp.float32
    dot = lambda a, b: jnp.dot(a, b, preferred_element_type=f32,
                               precision=jax.lax.Precision.DEFAULT)
    h = dot(rk_ref[:], Wk_ref[:])
    h = h + dot(aux_ref[:], Wa_ref[:])
    h = h + dot(res_ref[:], Wr_ref[:])
    h = h + b0_ref[:]
    h = _layernorm(h, gin_ref[:], bin_ref[:])
    h = jax.nn.gelu(dot(h, W1_ref[:]) + b1_ref[:])
    h = _layernorm(h, g1_ref[:], bb1_ref[:])
    h = jax.nn.gelu(dot(h, W2_ref[:]) + b2_ref[:])
    h = _layernorm(h, g2_ref[:], bb2_ref[:])
    logits = dot(h, Whd_ref[:]) + bhd_ref[:]

    T = out_ref.shape[0]
    E = out_ref.shape[1]
    G = role_ref.shape[0]
    ridg = jax.lax.broadcasted_iota(jnp.int32, (T, G), 0)
    cidg = jax.lax.broadcasted_iota(jnp.int32, (T, G), 1)
    q = (ridg // 128 == cidg).astype(f32)
    part = jnp.dot(q, role_ref[:], preferred_element_type=f32)
    rid = jax.lax.broadcasted_iota(jnp.int32, (T, 128), 0)
    cid = jax.lax.broadcasted_iota(jnp.int32, (T, 128), 1)
    lane = (rid % 128 == cid).astype(f32)
    col = jnp.dot(part * lane, jnp.ones((128, 1), f32),
                  preferred_element_type=f32)
    out_ref[:] = jnp.where(col == _ROLE_W, logits[:, :E], logits[:, E:])


def kernel(routing_key, aux_features, residual, role, Wk, bk, Wa, ba, Wr,
           br, g_in, b_in, W1, b1, g1, bb1, W2, b2, g2, bb2, Ww, bw, Wq,
           bq):
    N, KD = routing_key.shape
    D = aux_features.shape[1]
    H = Wk.shape[1]
    E = Ww.shape[1]
    T = _TILE
    G = T // 128

    rolef = role.astype(jnp.float32).reshape(N // 128, 128)
    b0 = (bk + ba + 0.1 * br).reshape(1, H)
    Whd = jnp.concatenate([Ww, Wq], axis=1)
    bhd = jnp.concatenate([bw, bq]).reshape(1, 2 * E)
    row = lambda v: v.reshape(1, -1)

    tok = lambda i: (i, 0)
    fix = lambda i: (0, 0)

    return pl.pallas_call(
        _fused,
        grid=(N // T,),
        in_specs=[
            pl.BlockSpec((T, KD), tok),
            pl.BlockSpec((T, D), tok),
            pl.BlockSpec((T, D), tok),
            pl.BlockSpec((G, 128), tok),
            pl.BlockSpec((KD, H), fix),
            pl.BlockSpec((D, H), fix),
            pl.BlockSpec((D, H), fix),
            pl.BlockSpec((1, H), fix),
            pl.BlockSpec((1, H), fix),
            pl.BlockSpec((1, H), fix),
            pl.BlockSpec((H, H), fix),
            pl.BlockSpec((1, H), fix),
            pl.BlockSpec((1, H), fix),
            pl.BlockSpec((1, H), fix),
            pl.BlockSpec((H, H), fix),
            pl.BlockSpec((1, H), fix),
            pl.BlockSpec((1, H), fix),
            pl.BlockSpec((1, H), fix),
            pl.BlockSpec((H, 2 * E), fix),
            pl.BlockSpec((1, 2 * E), fix),
        ],
        out_specs=pl.BlockSpec((T, E), tok),
        out_shape=jax.ShapeDtypeStruct((N, E), jnp.float32),
        compiler_params=pltpu.CompilerParams(
            dimension_semantics=("parallel",)),
    )(routing_key, aux_features, residual, rolef, Wk, Wa, 0.1 * Wr, b0,
      row(g_in), row(b_in), W1, row(b1), row(g1), row(bb1), W2, row(b2),
      row(g2), row(bb2), Whd, bhd)
